# Initial kernel scaffold; baseline (speedup 1.0000x reference)
import jax
import jax.numpy as jnp
from jax.experimental import pallas as pl


def kernel(hidden_states, gate_w, w1, w2):
    raise NotImplementedError("write your pallas kernel here")


# speedup vs baseline: 2.4736x; 2.4736x over previous
import functools

import jax
import jax.numpy as jnp
from jax.experimental import pallas as pl
from jax.experimental.pallas import tpu as pltpu

HIDDEN = 1024
INTER = 512
NUM_EXPERTS = 16
TOP_K = 2
S = 2048

NEG = -1e30


def _moe_dense_kernel(flat_ref, gate_w_ref, w1_ref, w2_ref,
                      out_ref, logits_ref, combine_ref):
    e = pl.program_id(0)
    x = flat_ref[...]

    @pl.when(e == 0)
    def _router():
        logits = jax.lax.dot_general(
            x, gate_w_ref[...], (((1,), (1,)), ((), ())),
            preferred_element_type=jnp.float32)
        logits_ref[...] = logits
        m = jnp.max(logits, axis=-1, keepdims=True)
        ex = jnp.exp(logits - m)
        probs = ex / jnp.sum(ex, axis=-1, keepdims=True)
        eids = jax.lax.broadcasted_iota(jnp.int32, probs.shape, 1)
        p0 = jnp.max(probs, axis=-1, keepdims=True)
        i0 = jnp.min(jnp.where(probs == p0, eids, NUM_EXPERTS), axis=-1,
                     keepdims=True)
        probs1 = jnp.where(eids == i0, NEG, probs)
        p1 = jnp.max(probs1, axis=-1, keepdims=True)
        i1 = jnp.min(jnp.where(probs1 == p1, eids, NUM_EXPERTS), axis=-1,
                     keepdims=True)
        denom = p0 + p1
        w0 = p0 / denom
        w1n = p1 / denom
        combine_ref[...] = (jnp.where(eids == i0, w0, 0.0)
                            + jnp.where(eids == i1, w1n, 0.0))

    gate_up = jax.lax.dot_general(
        x, w1_ref[0], (((1,), (1,)), ((), ())),
        preferred_element_type=jnp.float32)
    gate = gate_up[:, :INTER]
    up = gate_up[:, INTER:]
    act = (gate / (1.0 + jnp.exp(-gate))) * up
    expert_out = jax.lax.dot_general(
        act, w2_ref[0], (((1,), (1,)), ((), ())),
        preferred_element_type=jnp.float32)
    comb = combine_ref[...]
    ids = jax.lax.broadcasted_iota(jnp.int32, comb.shape, 1)
    cw = jnp.sum(jnp.where(ids == e, comb, 0.0), axis=-1, keepdims=True)
    contrib = cw * expert_out

    @pl.when(e == 0)
    def _init():
        out_ref[...] = contrib

    @pl.when(e != 0)
    def _acc():
        out_ref[...] += contrib


@jax.jit
def kernel(hidden_states, gate_w, w1, w2):
    flat = hidden_states.reshape(-1, HIDDEN)
    out, logits = pl.pallas_call(
        _moe_dense_kernel,
        grid=(NUM_EXPERTS,),
        in_specs=[
            pl.BlockSpec((S, HIDDEN), lambda e: (0, 0)),
            pl.BlockSpec((NUM_EXPERTS, HIDDEN), lambda e: (0, 0)),
            pl.BlockSpec((1, 2 * INTER, HIDDEN), lambda e: (e, 0, 0)),
            pl.BlockSpec((1, HIDDEN, INTER), lambda e: (e, 0, 0)),
        ],
        out_specs=[
            pl.BlockSpec((S, HIDDEN), lambda e: (0, 0)),
            pl.BlockSpec((S, NUM_EXPERTS), lambda e: (0, 0)),
        ],
        out_shape=[
            jax.ShapeDtypeStruct((S, HIDDEN), jnp.float32),
            jax.ShapeDtypeStruct((S, NUM_EXPERTS), jnp.float32),
        ],
        scratch_shapes=[pltpu.VMEM((S, NUM_EXPERTS), jnp.float32)],
    )(flat, gate_w, w1, w2)
    return out, logits
